# baseline (speedup 1.0000x reference)
import functools

import jax
import jax.numpy as jnp
from jax.experimental import pallas as pl
from jax.experimental.pallas import tpu as pltpu

NUM_EXPERTS = 8
IN_FEATURES = 1024
OUT_FEATURES = 1024
N_TOKENS = 8192
BLOCK_N = 2048


def _moe_body(x_ref, gw_ref, gb_ref, ew_ref, eb_ref, out_ref, s_ref):
    e = pl.program_id(1)
    x = x_ref[...]

    @pl.when(e == 0)
    def _init():
        logits = (
            jnp.dot(x, gw_ref[...], preferred_element_type=jnp.float32)
            + gb_ref[...]
        )
        m = jnp.max(logits, axis=-1, keepdims=True)
        ex = jnp.exp(logits - m)
        s = ex / jnp.sum(ex, axis=-1, keepdims=True)
        s_ref[...] = s
        out_ref[...] = jnp.dot(s, eb_ref[...], preferred_element_type=jnp.float32)

    s = s_ref[...]
    lane = jax.lax.broadcasted_iota(jnp.int32, s.shape, 1)
    col = jnp.sum(jnp.where(lane == e, s, 0.0), axis=-1, keepdims=True)
    xs = (col * x).astype(jnp.bfloat16)
    out_ref[...] += jnp.dot(xs, ew_ref[0], preferred_element_type=jnp.float32)


@jax.jit
def kernel(x, gate_W, gate_b, expert_W, expert_b):
    n_blocks = N_TOKENS // BLOCK_N
    grid = (n_blocks, NUM_EXPERTS)
    out = pl.pallas_call(
        _moe_body,
        grid=grid,
        in_specs=[
            pl.BlockSpec((BLOCK_N, IN_FEATURES), lambda i, e: (i, 0)),
            pl.BlockSpec((IN_FEATURES, NUM_EXPERTS), lambda i, e: (0, 0)),
            pl.BlockSpec((1, NUM_EXPERTS), lambda i, e: (0, 0)),
            pl.BlockSpec((1, IN_FEATURES, OUT_FEATURES), lambda i, e: (e, 0, 0)),
            pl.BlockSpec((NUM_EXPERTS, OUT_FEATURES), lambda i, e: (0, 0)),
        ],
        out_specs=pl.BlockSpec((BLOCK_N, OUT_FEATURES), lambda i, e: (i, 0)),
        out_shape=jax.ShapeDtypeStruct((N_TOKENS, OUT_FEATURES), jnp.float32),
        scratch_shapes=[pltpu.VMEM((BLOCK_N, NUM_EXPERTS), jnp.float32)],
        compiler_params=pltpu.CompilerParams(
            dimension_semantics=("parallel", "arbitrary"),
        ),
    )(x, gate_W, gate_b.reshape(1, NUM_EXPERTS), expert_W.astype(jnp.bfloat16), expert_b)
    return out


# speedup vs baseline: 1.1065x; 1.1065x over previous
import functools

import jax
import jax.numpy as jnp
from jax.experimental import pallas as pl
from jax.experimental.pallas import tpu as pltpu

NUM_EXPERTS = 8
IN_FEATURES = 1024
OUT_FEATURES = 1024
N_TOKENS = 8192
BLOCK_N = 2048


def _moe_body(x_ref, gw_ref, gb_ref, ew_ref, eb_ref, out_ref, s_ref):
    e = pl.program_id(1)
    x = x_ref[...]

    @pl.when(e == 0)
    def _init():
        logits = (
            jnp.dot(x, gw_ref[...], preferred_element_type=jnp.float32)
            + gb_ref[...]
        )
        m = jnp.max(logits, axis=-1, keepdims=True)
        ex = jnp.exp(logits - m)
        s = ex / jnp.sum(ex, axis=-1, keepdims=True)
        s_ref[...] = s
        out_ref[...] = jnp.dot(s, eb_ref[...], preferred_element_type=jnp.float32)

    s = s_ref[...]
    lane = jax.lax.broadcasted_iota(jnp.int32, s.shape, 1)
    col = jnp.sum(jnp.where(lane == e, s, 0.0), axis=-1, keepdims=True)
    out_ref[...] += jnp.dot(col * x, ew_ref[0], preferred_element_type=jnp.float32)


@jax.jit
def kernel(x, gate_W, gate_b, expert_W, expert_b):
    n_blocks = N_TOKENS // BLOCK_N
    grid = (n_blocks, NUM_EXPERTS)
    out = pl.pallas_call(
        _moe_body,
        grid=grid,
        in_specs=[
            pl.BlockSpec((BLOCK_N, IN_FEATURES), lambda i, e: (i, 0)),
            pl.BlockSpec((IN_FEATURES, NUM_EXPERTS), lambda i, e: (0, 0)),
            pl.BlockSpec((1, NUM_EXPERTS), lambda i, e: (0, 0)),
            pl.BlockSpec((1, IN_FEATURES, OUT_FEATURES), lambda i, e: (e, 0, 0)),
            pl.BlockSpec((NUM_EXPERTS, OUT_FEATURES), lambda i, e: (0, 0)),
        ],
        out_specs=pl.BlockSpec((BLOCK_N, OUT_FEATURES), lambda i, e: (i, 0)),
        out_shape=jax.ShapeDtypeStruct((N_TOKENS, OUT_FEATURES), jnp.float32),
        scratch_shapes=[pltpu.VMEM((BLOCK_N, NUM_EXPERTS), jnp.float32)],
        compiler_params=pltpu.CompilerParams(
            dimension_semantics=("parallel", "arbitrary"),
        ),
    )(x, gate_W, gate_b.reshape(1, NUM_EXPERTS), expert_W, expert_b)
    return out


# speedup vs baseline: 1.1533x; 1.0422x over previous
import jax
import jax.numpy as jnp
from jax.experimental import pallas as pl
from jax.experimental.pallas import tpu as pltpu

NUM_EXPERTS = 8
IN_FEATURES = 1024
OUT_FEATURES = 1024
N_TOKENS = 8192
BLOCK_N = 1024


def _moe_body(x_ref, gw_ref, gb_ref, ew_ref, eb_ref, out_ref, xs_ref):
    x = x_ref[...]
    logits = (
        jnp.dot(x, gw_ref[...], preferred_element_type=jnp.float32) + gb_ref[...]
    )
    m = jnp.max(logits, axis=-1, keepdims=True)
    ex = jnp.exp(logits - m)
    s = ex / jnp.sum(ex, axis=-1, keepdims=True)
    for e in range(NUM_EXPERTS):
        xs_ref[:, e * IN_FEATURES : (e + 1) * IN_FEATURES] = (
            s[:, e : e + 1] * x
        ).astype(jnp.bfloat16)
    out_ref[...] = jnp.dot(
        xs_ref[...], ew_ref[...], preferred_element_type=jnp.float32
    ) + jnp.dot(s, eb_ref[...], preferred_element_type=jnp.float32)


@jax.jit
def kernel(x, gate_W, gate_b, expert_W, expert_b):
    n_blocks = N_TOKENS // BLOCK_N
    ew = expert_W.reshape(NUM_EXPERTS * IN_FEATURES, OUT_FEATURES).astype(
        jnp.bfloat16
    )
    out = pl.pallas_call(
        _moe_body,
        grid=(n_blocks,),
        in_specs=[
            pl.BlockSpec((BLOCK_N, IN_FEATURES), lambda i: (i, 0)),
            pl.BlockSpec((IN_FEATURES, NUM_EXPERTS), lambda i: (0, 0)),
            pl.BlockSpec((1, NUM_EXPERTS), lambda i: (0, 0)),
            pl.BlockSpec((NUM_EXPERTS * IN_FEATURES, OUT_FEATURES), lambda i: (0, 0)),
            pl.BlockSpec((NUM_EXPERTS, OUT_FEATURES), lambda i: (0, 0)),
        ],
        out_specs=pl.BlockSpec((BLOCK_N, OUT_FEATURES), lambda i: (i, 0)),
        out_shape=jax.ShapeDtypeStruct((N_TOKENS, OUT_FEATURES), jnp.float32),
        scratch_shapes=[
            pltpu.VMEM((BLOCK_N, NUM_EXPERTS * IN_FEATURES), jnp.bfloat16)
        ],
        compiler_params=pltpu.CompilerParams(
            dimension_semantics=("arbitrary",),
        ),
    )(x, gate_W, gate_b.reshape(1, NUM_EXPERTS), ew, expert_b)
    return out
